# Initial kernel scaffold; baseline (speedup 1.0000x reference)
import jax
import jax.numpy as jnp
from jax.experimental import pallas as pl


def kernel(x, W1, b1, g1, beta1, W2, b2, g2, beta2, W3, b3, g3, beta3):
    raise NotImplementedError("write your pallas kernel here")


# speedup vs baseline: 1.7068x; 1.7068x over previous
import jax
import jax.numpy as jnp
from jax.experimental import pallas as pl
from jax.experimental.pallas import tpu as pltpu

_B = 128
_CH = 8
_F = 32
_WL = 0.638
_PITCH = 8e-06
_H1 = 512
_D1 = 768
_D2 = 1536

_G_MLP = 2
_G_EXP = 16


def _freq_bands():
    wavelength = _WL * 1e-06
    min_fre = 2 * jnp.pi / wavelength * (1 - 2 * (wavelength / _PITCH / 2) ** 2) ** 0.5
    max_fre = 2 * jnp.pi / wavelength
    fb = (max_fre - min_fre) / _F * jnp.linspace(1.0, _F, _F) + min_fre
    return fb.astype(jnp.float32).reshape(1, _F)


def _bin_onehot(length):
    ax = jnp.linspace(-float(length), float(length), 2 * length)
    xg, yg = jnp.meshgrid(ax, ax, indexing='ij')
    dis = jnp.sqrt(xg ** 2 + yg ** 2)
    interval = jnp.max(dis) / length
    idx = jnp.floor(dis / (interval + 0.0001)).astype(jnp.int32)
    flat = idx.reshape(1, 4 * length * length)
    lanes = jnp.arange(length, dtype=jnp.int32).reshape(length, 1)
    return (flat == lanes).astype(jnp.float32)


def _ln_scale(z, g, b):
    m = jnp.mean(z, axis=-1, keepdims=True)
    v = jnp.mean((z - m) ** 2, axis=-1, keepdims=True)
    return (z - m) * jax.lax.rsqrt(v + 1e-5) * g + b


def _mlp_body(x_ref, fb_ref, w1c_ref, w1s_ref, b1_ref, g1_ref, be1_ref,
              w2_ref, b2_ref, g2_ref, be2_ref,
              w3_ref, b3_ref, g3_ref, be3_ref, kv_ref, kvb_ref):
    ds = x_ref[...] * fb_ref[...]
    z1 = (jnp.dot(jnp.cos(ds), w1c_ref[...], preferred_element_type=jnp.float32)
          + jnp.dot(jnp.sin(ds), w1s_ref[...], preferred_element_type=jnp.float32)
          + b1_ref[...])
    h = jnp.tanh(_ln_scale(z1, g1_ref[...], be1_ref[...]))
    z2 = jnp.dot(h, w2_ref[...], preferred_element_type=jnp.float32) + b2_ref[...]
    kv = jnp.tanh(_ln_scale(z2, g2_ref[...], be2_ref[...]))
    kv_ref[...] = kv
    z3 = jnp.dot(kv, w3_ref[...], preferred_element_type=jnp.float32) + b3_ref[...]
    kvb_ref[...] = jnp.tanh(_ln_scale(z3, g3_ref[...], be3_ref[...]))


def _expand_body(k1_ref, k2_ref, k3_ref, kv1_ref, kv2_ref, kv3_ref,
                 oh64_ref, oh32_ref, oh16_ref,
                 o1_ref, o2_ref, o3_ref, o11_ref, o22_ref, o33_ref):
    oh64 = oh64_ref[...]
    oh32 = oh32_ref[...]
    oh16 = oh16_ref[...]
    o1_ref[...] = jnp.dot(k1_ref[...], oh64, preferred_element_type=jnp.float32)
    o2_ref[...] = jnp.dot(k2_ref[...], oh32, preferred_element_type=jnp.float32)
    o3_ref[...] = jnp.dot(k3_ref[...], oh16, preferred_element_type=jnp.float32)
    o11_ref[...] = jnp.dot(kv1_ref[...], oh64, preferred_element_type=jnp.float32)
    o22_ref[...] = jnp.dot(kv2_ref[...], oh32, preferred_element_type=jnp.float32)
    o33_ref[...] = jnp.dot(kv3_ref[...], oh16, preferred_element_type=jnp.float32)


def _row_spec(rows, cols, grid):
    return pl.BlockSpec((rows // grid, cols), lambda i: (i, 0))


def _full_spec(rows, cols):
    return pl.BlockSpec((rows, cols), lambda i: (0, 0))


def kernel(x, W1, b1, g1, beta1, W2, b2, g2, beta2, W3, b3, g3, beta3):
    f32 = jnp.float32
    x = x.astype(f32)
    fb = _freq_bands()
    w1c = W1[:_F, :]
    w1s = W1[_F:, :]
    row = lambda v: v.reshape(1, -1)

    kv, kvb = pl.pallas_call(
        _mlp_body,
        grid=(_G_MLP,),
        in_specs=[
            pl.BlockSpec((_B // _G_MLP, 1), lambda i: (i, 0)),
            _full_spec(1, _F),
            _full_spec(_F, _H1), _full_spec(_F, _H1),
            _full_spec(1, _H1), _full_spec(1, _H1), _full_spec(1, _H1),
            _full_spec(_H1, _D1),
            _full_spec(1, _D1), _full_spec(1, _D1), _full_spec(1, _D1),
            _full_spec(_D1, _D2),
            _full_spec(1, _D2), _full_spec(1, _D2), _full_spec(1, _D2),
        ],
        out_specs=[
            pl.BlockSpec((_B // _G_MLP, _D1), lambda i: (i, 0)),
            pl.BlockSpec((_B // _G_MLP, _D2), lambda i: (i, 0)),
        ],
        out_shape=[
            jax.ShapeDtypeStruct((_B, _D1), f32),
            jax.ShapeDtypeStruct((_B, _D2), f32),
        ],
        compiler_params=pltpu.CompilerParams(
            dimension_semantics=("parallel",),
        ),
        name="distance_mlp",
    )(x, fb, w1c, w1s, row(b1), row(g1), row(beta1),
      W2, row(b2), row(g2), row(beta2),
      W3, row(b3), row(g3), row(beta3))

    kv1 = kv[:, :256].reshape(_B * 4, 64)
    kv2 = kv[:, 256:512].reshape(_B * 8, 32)
    kv3 = kv[:, 512:].reshape(_B * 16, 16)
    k1 = kvb[:, :512].reshape(_B * 8, 64)
    k2 = kvb[:, 512:1024].reshape(_B * 16, 32)
    k3 = kvb[:, 1024:].reshape(_B * 32, 16)

    g = _G_EXP
    o1, o2, o3, o11, o22, o33 = pl.pallas_call(
        _expand_body,
        grid=(g,),
        in_specs=[
            _row_spec(_B * 8, 64, g),
            _row_spec(_B * 16, 32, g),
            _row_spec(_B * 32, 16, g),
            _row_spec(_B * 4, 64, g),
            _row_spec(_B * 8, 32, g),
            _row_spec(_B * 16, 16, g),
            _full_spec(64, 16384),
            _full_spec(32, 4096),
            _full_spec(16, 1024),
        ],
        out_specs=[
            _row_spec(_B * 8, 16384, g),
            _row_spec(_B * 16, 4096, g),
            _row_spec(_B * 32, 1024, g),
            _row_spec(_B * 4, 16384, g),
            _row_spec(_B * 8, 4096, g),
            _row_spec(_B * 16, 1024, g),
        ],
        out_shape=[
            jax.ShapeDtypeStruct((_B * 8, 16384), f32),
            jax.ShapeDtypeStruct((_B * 16, 4096), f32),
            jax.ShapeDtypeStruct((_B * 32, 1024), f32),
            jax.ShapeDtypeStruct((_B * 4, 16384), f32),
            jax.ShapeDtypeStruct((_B * 8, 4096), f32),
            jax.ShapeDtypeStruct((_B * 16, 1024), f32),
        ],
        compiler_params=pltpu.CompilerParams(
            dimension_semantics=("parallel",),
            vmem_limit_bytes=56 * 1024 * 1024,
        ),
        name="distance_expand",
    )(k1, k2, k3, kv1, kv2, kv3,
      _bin_onehot(64), _bin_onehot(32), _bin_onehot(16))

    c1 = o1.reshape(_B, 8, 128, 128)
    c2 = o2.reshape(_B, 16, 64, 64)
    c3 = o3.reshape(_B, 32, 32, 32)
    c11 = o11.reshape(_B, 4, 128, 128)
    c22 = o22.reshape(_B, 8, 64, 64)
    c33 = o33.reshape(_B, 16, 32, 32)
    return (c1, c2, c3, c11, c22, c33)


# speedup vs baseline: 3.2435x; 1.9003x over previous
import jax
import jax.numpy as jnp
from jax.experimental import pallas as pl
from jax.experimental.pallas import tpu as pltpu

_B = 128
_F = 32
_WL = 0.638
_PITCH = 8e-06
_H1 = 512
_D1 = 768
_D2 = 1536

_G_MLP = 2
_G_EXP = 8


def _freq_bands():
    wavelength = _WL * 1e-06
    min_fre = 2 * jnp.pi / wavelength * (1 - 2 * (wavelength / _PITCH / 2) ** 2) ** 0.5
    max_fre = 2 * jnp.pi / wavelength
    fb = (max_fre - min_fre) / _F * jnp.linspace(1.0, _F, _F) + min_fre
    return fb.astype(jnp.float32).reshape(1, _F)


def _bin_onehot_t(length):
    ax = jnp.linspace(-float(length), float(length), 2 * length)
    xg, yg = jnp.meshgrid(ax, ax, indexing='ij')
    dis = jnp.sqrt(xg ** 2 + yg ** 2)
    interval = jnp.max(dis) / length
    idx = jnp.floor(dis / (interval + 0.0001)).astype(jnp.int32)
    flat = idx.reshape(4 * length * length, 1)
    lanes = jnp.arange(length, dtype=jnp.int32).reshape(1, length)
    return (flat == lanes).astype(jnp.float32)


def _ln_scale(z, g, b):
    m = jnp.mean(z, axis=-1, keepdims=True)
    v = jnp.mean((z - m) ** 2, axis=-1, keepdims=True)
    return (z - m) * jax.lax.rsqrt(v + 1e-5) * g + b


def _mlp_body(x_ref, fb_ref, w1c_ref, w1s_ref, b1_ref, g1_ref, be1_ref,
              w2_ref, b2_ref, g2_ref, be2_ref,
              w3_ref, b3_ref, g3_ref, be3_ref, kv_ref, kvb_ref):
    ds = x_ref[...] * fb_ref[...]
    z1 = (jnp.dot(jnp.cos(ds), w1c_ref[...], preferred_element_type=jnp.float32)
          + jnp.dot(jnp.sin(ds), w1s_ref[...], preferred_element_type=jnp.float32)
          + b1_ref[...])
    h = jnp.tanh(_ln_scale(z1, g1_ref[...], be1_ref[...]))
    z2 = jnp.dot(h, w2_ref[...], preferred_element_type=jnp.float32) + b2_ref[...]
    kv = jnp.tanh(_ln_scale(z2, g2_ref[...], be2_ref[...]))
    kv_ref[...] = kv
    z3 = jnp.dot(kv, w3_ref[...], preferred_element_type=jnp.float32) + b3_ref[...]
    kvb_ref[...] = jnp.tanh(_ln_scale(z3, g3_ref[...], be3_ref[...]))


def _expand_body(oh64_ref, oh32_ref, oh16_ref,
                 k1_ref, k2_ref, k3_ref, kv1_ref, kv2_ref, kv3_ref,
                 o1_ref, o2_ref, o3_ref, o11_ref, o22_ref, o33_ref):
    def expand(oh_ref, ct_ref, out_ref):
        n_ch, di = out_ref.shape[0], out_ref.shape[1]
        t = jnp.dot(oh_ref[...], ct_ref[...],
                    preferred_element_type=jnp.float32)
        for c in range(n_ch):
            out_ref[c] = t[:, c * _B:(c + 1) * _B].reshape(di, -1, _B)
    expand(oh64_ref, k1_ref, o1_ref)
    expand(oh32_ref, k2_ref, o2_ref)
    expand(oh16_ref, k3_ref, o3_ref)
    expand(oh64_ref, kv1_ref, o11_ref)
    expand(oh32_ref, kv2_ref, o22_ref)
    expand(oh16_ref, kv3_ref, o33_ref)


def _full_spec(rows, cols):
    return pl.BlockSpec((rows, cols), lambda i: (0, 0))


def kernel(x, W1, b1, g1, beta1, W2, b2, g2, beta2, W3, b3, g3, beta3):
    f32 = jnp.float32
    x = x.astype(f32)
    fb = _freq_bands()
    w1c = W1[:_F, :]
    w1s = W1[_F:, :]
    row = lambda v: v.reshape(1, -1)

    kv, kvb = pl.pallas_call(
        _mlp_body,
        grid=(_G_MLP,),
        in_specs=[
            pl.BlockSpec((_B // _G_MLP, 1), lambda i: (i, 0)),
            _full_spec(1, _F),
            _full_spec(_F, _H1), _full_spec(_F, _H1),
            _full_spec(1, _H1), _full_spec(1, _H1), _full_spec(1, _H1),
            _full_spec(_H1, _D1),
            _full_spec(1, _D1), _full_spec(1, _D1), _full_spec(1, _D1),
            _full_spec(_D1, _D2),
            _full_spec(1, _D2), _full_spec(1, _D2), _full_spec(1, _D2),
        ],
        out_specs=[
            pl.BlockSpec((_B // _G_MLP, _D1), lambda i: (i, 0)),
            pl.BlockSpec((_B // _G_MLP, _D2), lambda i: (i, 0)),
        ],
        out_shape=[
            jax.ShapeDtypeStruct((_B, _D1), f32),
            jax.ShapeDtypeStruct((_B, _D2), f32),
        ],
        compiler_params=pltpu.CompilerParams(
            dimension_semantics=("parallel",),
        ),
        name="distance_mlp",
    )(x, fb, w1c, w1s, row(b1), row(g1), row(beta1),
      W2, row(b2), row(g2), row(beta2),
      W3, row(b3), row(g3), row(beta3))

    def comps_t(mat, n_ch, length):
        return (mat.reshape(_B, n_ch, length)
                   .transpose(2, 1, 0)
                   .reshape(length, n_ch * _B))

    kv1 = comps_t(kv[:, :256], 4, 64)
    kv2 = comps_t(kv[:, 256:512], 8, 32)
    kv3 = comps_t(kv[:, 512:], 16, 16)
    k1 = comps_t(kvb[:, :512], 8, 64)
    k2 = comps_t(kvb[:, 512:1024], 16, 32)
    k3 = comps_t(kvb[:, 1024:], 32, 16)

    g = _G_EXP
    r64, r32, r16 = 16384 // g, 4096 // g, 1024 // g
    o1, o2, o3, o11, o22, o33 = pl.pallas_call(
        _expand_body,
        grid=(g,),
        in_specs=[
            pl.BlockSpec((r64, 64), lambda i: (i, 0)),
            pl.BlockSpec((r32, 32), lambda i: (i, 0)),
            pl.BlockSpec((r16, 16), lambda i: (i, 0)),
            _full_spec(64, 8 * _B),
            _full_spec(32, 16 * _B),
            _full_spec(16, 32 * _B),
            _full_spec(64, 4 * _B),
            _full_spec(32, 8 * _B),
            _full_spec(16, 16 * _B),
        ],
        out_specs=[
            pl.BlockSpec((8, 128 // g, 128, _B), lambda i: (0, i, 0, 0)),
            pl.BlockSpec((16, 64 // g, 64, _B), lambda i: (0, i, 0, 0)),
            pl.BlockSpec((32, 32 // g, 32, _B), lambda i: (0, i, 0, 0)),
            pl.BlockSpec((4, 128 // g, 128, _B), lambda i: (0, i, 0, 0)),
            pl.BlockSpec((8, 64 // g, 64, _B), lambda i: (0, i, 0, 0)),
            pl.BlockSpec((16, 32 // g, 32, _B), lambda i: (0, i, 0, 0)),
        ],
        out_shape=[
            jax.ShapeDtypeStruct((8, 128, 128, _B), f32),
            jax.ShapeDtypeStruct((16, 64, 64, _B), f32),
            jax.ShapeDtypeStruct((32, 32, 32, _B), f32),
            jax.ShapeDtypeStruct((4, 128, 128, _B), f32),
            jax.ShapeDtypeStruct((8, 64, 64, _B), f32),
            jax.ShapeDtypeStruct((16, 32, 32, _B), f32),
        ],
        compiler_params=pltpu.CompilerParams(
            dimension_semantics=("parallel",),
            vmem_limit_bytes=56 * 1024 * 1024,
        ),
        name="distance_expand",
    )(_bin_onehot_t(64), _bin_onehot_t(32), _bin_onehot_t(16),
      k1, k2, k3, kv1, kv2, kv3)

    def finalize(o):
        return o.transpose(3, 0, 1, 2)

    c1 = finalize(o1)
    c2 = finalize(o2)
    c3 = finalize(o3)
    c11 = finalize(o11)
    c22 = finalize(o22)
    c33 = finalize(o33)
    return (c1, c2, c3, c11, c22, c33)


# speedup vs baseline: 3.6287x; 1.1188x over previous
import jax
import jax.numpy as jnp
from jax.experimental import pallas as pl
from jax.experimental.pallas import tpu as pltpu

_B = 128
_F = 32
_WL = 0.638
_PITCH = 8e-06
_H1 = 512
_D1 = 768
_D2 = 1536

_G_MLP = 2
_G_EXP = 8


def _freq_bands():
    wavelength = _WL * 1e-06
    min_fre = 2 * jnp.pi / wavelength * (1 - 2 * (wavelength / _PITCH / 2) ** 2) ** 0.5
    max_fre = 2 * jnp.pi / wavelength
    fb = (max_fre - min_fre) / _F * jnp.linspace(1.0, _F, _F) + min_fre
    return fb.astype(jnp.float32).reshape(1, _F)


def _bin_onehot_t(length):
    ax = jnp.linspace(-float(length), float(length), 2 * length)
    xg, yg = jnp.meshgrid(ax, ax, indexing='ij')
    dis = jnp.sqrt(xg ** 2 + yg ** 2)
    interval = jnp.max(dis) / length
    idx = jnp.floor(dis / (interval + 0.0001)).astype(jnp.int32)
    flat = idx.reshape(4 * length * length, 1)
    lanes = jnp.arange(length, dtype=jnp.int32).reshape(1, length)
    return (flat == lanes).astype(jnp.float32)


def _ln_scale(z, g, b):
    m = jnp.mean(z, axis=-1, keepdims=True)
    v = jnp.mean((z - m) ** 2, axis=-1, keepdims=True)
    return (z - m) * jax.lax.rsqrt(v + 1e-5) * g + b


def _mlp_body(x_ref, fb_ref, w1c_ref, w1s_ref, b1_ref, g1_ref, be1_ref,
              w2_ref, b2_ref, g2_ref, be2_ref,
              w3_ref, b3_ref, g3_ref, be3_ref, kv_ref, kvb_ref):
    ds = x_ref[...] * fb_ref[...]
    z1 = (jnp.dot(jnp.cos(ds), w1c_ref[...], preferred_element_type=jnp.float32)
          + jnp.dot(jnp.sin(ds), w1s_ref[...], preferred_element_type=jnp.float32)
          + b1_ref[...])
    h = jnp.tanh(_ln_scale(z1, g1_ref[...], be1_ref[...]))
    z2 = jnp.dot(h, w2_ref[...], preferred_element_type=jnp.float32) + b2_ref[...]
    kv = jnp.tanh(_ln_scale(z2, g2_ref[...], be2_ref[...]))
    kv_ref[...] = kv
    z3 = jnp.dot(kv, w3_ref[...], preferred_element_type=jnp.float32) + b3_ref[...]
    kvb_ref[...] = jnp.tanh(_ln_scale(z3, g3_ref[...], be3_ref[...]))


def _expand(oh_ref, ct_ref, out_ref):
    n_ch, di = out_ref.shape[0], out_ref.shape[1]
    t = jnp.dot(oh_ref[...], ct_ref[...],
                preferred_element_type=jnp.float32)
    for c in range(n_ch):
        out_ref[c] = t[:, c * _B:(c + 1) * _B].reshape(di, -1, _B)


def _expand_a_body(oh64_ref, k1_ref, kv1_ref, o1_ref, o11_ref):
    _expand(oh64_ref, k1_ref, o1_ref)
    _expand(oh64_ref, kv1_ref, o11_ref)


def _expand_b_body(oh32_ref, oh16_ref, k2_ref, k3_ref, kv2_ref, kv3_ref,
                   o2_ref, o3_ref, o22_ref, o33_ref):
    _expand(oh32_ref, k2_ref, o2_ref)
    _expand(oh16_ref, k3_ref, o3_ref)
    _expand(oh32_ref, kv2_ref, o22_ref)
    _expand(oh16_ref, kv3_ref, o33_ref)


def _full_spec(rows, cols):
    return pl.BlockSpec((rows, cols), lambda i: (0, 0))


def kernel(x, W1, b1, g1, beta1, W2, b2, g2, beta2, W3, b3, g3, beta3):
    f32 = jnp.float32
    x = x.astype(f32)
    fb = _freq_bands()
    w1c = W1[:_F, :]
    w1s = W1[_F:, :]
    row = lambda v: v.reshape(1, -1)

    kv, kvb = pl.pallas_call(
        _mlp_body,
        grid=(_G_MLP,),
        in_specs=[
            pl.BlockSpec((_B // _G_MLP, 1), lambda i: (i, 0)),
            _full_spec(1, _F),
            _full_spec(_F, _H1), _full_spec(_F, _H1),
            _full_spec(1, _H1), _full_spec(1, _H1), _full_spec(1, _H1),
            _full_spec(_H1, _D1),
            _full_spec(1, _D1), _full_spec(1, _D1), _full_spec(1, _D1),
            _full_spec(_D1, _D2),
            _full_spec(1, _D2), _full_spec(1, _D2), _full_spec(1, _D2),
        ],
        out_specs=[
            pl.BlockSpec((_B // _G_MLP, _D1), lambda i: (i, 0)),
            pl.BlockSpec((_B // _G_MLP, _D2), lambda i: (i, 0)),
        ],
        out_shape=[
            jax.ShapeDtypeStruct((_B, _D1), f32),
            jax.ShapeDtypeStruct((_B, _D2), f32),
        ],
        compiler_params=pltpu.CompilerParams(
            dimension_semantics=("parallel",),
        ),
        name="distance_mlp",
    )(x, fb, w1c, w1s, row(b1), row(g1), row(beta1),
      W2, row(b2), row(g2), row(beta2),
      W3, row(b3), row(g3), row(beta3))

    def comps_t(mat, n_ch, length):
        return (mat.reshape(_B, n_ch, length)
                   .transpose(2, 1, 0)
                   .reshape(length, n_ch * _B))

    kv1 = comps_t(kv[:, :256], 4, 64)
    kv2 = comps_t(kv[:, 256:512], 8, 32)
    kv3 = comps_t(kv[:, 512:], 16, 16)
    k1 = comps_t(kvb[:, :512], 8, 64)
    k2 = comps_t(kvb[:, 512:1024], 16, 32)
    k3 = comps_t(kvb[:, 1024:], 32, 16)

    g = _G_EXP
    r64, r32, r16 = 16384 // g, 4096 // g, 1024 // g
    o1, o11 = pl.pallas_call(
        _expand_a_body,
        grid=(g,),
        in_specs=[
            pl.BlockSpec((r64, 64), lambda i: (i, 0)),
            _full_spec(64, 8 * _B),
            _full_spec(64, 4 * _B),
        ],
        out_specs=[
            pl.BlockSpec((8, 128 // g, 128, _B), lambda i: (0, i, 0, 0)),
            pl.BlockSpec((4, 128 // g, 128, _B), lambda i: (0, i, 0, 0)),
        ],
        out_shape=[
            jax.ShapeDtypeStruct((8, 128, 128, _B), f32),
            jax.ShapeDtypeStruct((4, 128, 128, _B), f32),
        ],
        compiler_params=pltpu.CompilerParams(
            dimension_semantics=("parallel",),
            vmem_limit_bytes=48 * 1024 * 1024,
        ),
        name="distance_expand_a",
    )(_bin_onehot_t(64), k1, kv1)

    o2, o3, o22, o33 = pl.pallas_call(
        _expand_b_body,
        grid=(g,),
        in_specs=[
            pl.BlockSpec((r32, 32), lambda i: (i, 0)),
            pl.BlockSpec((r16, 16), lambda i: (i, 0)),
            _full_spec(32, 16 * _B),
            _full_spec(16, 32 * _B),
            _full_spec(32, 8 * _B),
            _full_spec(16, 16 * _B),
        ],
        out_specs=[
            pl.BlockSpec((16, 64 // g, 64, _B), lambda i: (0, i, 0, 0)),
            pl.BlockSpec((32, 32 // g, 32, _B), lambda i: (0, i, 0, 0)),
            pl.BlockSpec((8, 64 // g, 64, _B), lambda i: (0, i, 0, 0)),
            pl.BlockSpec((16, 32 // g, 32, _B), lambda i: (0, i, 0, 0)),
        ],
        out_shape=[
            jax.ShapeDtypeStruct((16, 64, 64, _B), f32),
            jax.ShapeDtypeStruct((32, 32, 32, _B), f32),
            jax.ShapeDtypeStruct((8, 64, 64, _B), f32),
            jax.ShapeDtypeStruct((16, 32, 32, _B), f32),
        ],
        compiler_params=pltpu.CompilerParams(
            dimension_semantics=("parallel",),
            vmem_limit_bytes=48 * 1024 * 1024,
        ),
        name="distance_expand_b",
    )(_bin_onehot_t(32), _bin_onehot_t(16), k2, k3, kv2, kv3)

    def finalize(o):
        return o.transpose(3, 0, 1, 2)

    c1 = finalize(o1)
    c2 = finalize(o2)
    c3 = finalize(o3)
    c11 = finalize(o11)
    c22 = finalize(o22)
    c33 = finalize(o33)
    return (c1, c2, c3, c11, c22, c33)


# speedup vs baseline: 5.0079x; 1.3801x over previous
import jax
import jax.numpy as jnp
from jax.experimental import pallas as pl
from jax.experimental.pallas import tpu as pltpu

_B = 128
_F = 32
_WL = 0.638
_PITCH = 8e-06
_H1 = 512
_D1 = 768
_D2 = 1536

_G_MLP = 2
_G_EXP = 8


def _freq_bands():
    wavelength = _WL * 1e-06
    min_fre = 2 * jnp.pi / wavelength * (1 - 2 * (wavelength / _PITCH / 2) ** 2) ** 0.5
    max_fre = 2 * jnp.pi / wavelength
    fb = (max_fre - min_fre) / _F * jnp.linspace(1.0, _F, _F) + min_fre
    return fb.astype(jnp.float32).reshape(1, _F)


def _bin_idx(length):
    ax = jnp.linspace(-float(length), float(length), 2 * length)
    xg, yg = jnp.meshgrid(ax, ax, indexing='ij')
    dis = jnp.sqrt(xg ** 2 + yg ** 2)
    interval = jnp.max(dis) / length
    return jnp.floor(dis / (interval + 0.0001)).astype(jnp.int32)


def _bin_onehot_t(length):
    idx = _bin_idx(length)
    flat = idx.reshape(4 * length * length, 1)
    lanes = jnp.arange(length, dtype=jnp.int32).reshape(1, length)
    return (flat == lanes).astype(jnp.float32)


def _ln_scale(z, g, b):
    m = jnp.mean(z, axis=-1, keepdims=True)
    v = jnp.mean((z - m) ** 2, axis=-1, keepdims=True)
    return (z - m) * jax.lax.rsqrt(v + 1e-5) * g + b


def _mlp_body(x_ref, fb_ref, w1c_ref, w1s_ref, b1_ref, g1_ref, be1_ref,
              w2_ref, b2_ref, g2_ref, be2_ref,
              w3_ref, b3_ref, g3_ref, be3_ref, kv_ref, kvb_ref):
    ds = x_ref[...] * fb_ref[...]
    z1 = (jnp.dot(jnp.cos(ds), w1c_ref[...], preferred_element_type=jnp.float32)
          + jnp.dot(jnp.sin(ds), w1s_ref[...], preferred_element_type=jnp.float32)
          + b1_ref[...])
    h = jnp.tanh(_ln_scale(z1, g1_ref[...], be1_ref[...]))
    z2 = jnp.dot(h, w2_ref[...], preferred_element_type=jnp.float32) + b2_ref[...]
    kv = jnp.tanh(_ln_scale(z2, g2_ref[...], be2_ref[...]))
    kv_ref[...] = kv
    z3 = jnp.dot(kv, w3_ref[...], preferred_element_type=jnp.float32) + b3_ref[...]
    kvb_ref[...] = jnp.tanh(_ln_scale(z3, g3_ref[...], be3_ref[...]))


def _expand(oh_ref, ct_ref, out_ref):
    n_ch, di = out_ref.shape[0], out_ref.shape[1]
    t = jnp.dot(oh_ref[...], ct_ref[...],
                preferred_element_type=jnp.float32)
    for c in range(n_ch):
        out_ref[c] = t[:, c * _B:(c + 1) * _B].reshape(di, -1, _B)


def _expand_a_body(idx_ref, k1_ref, kv1_ref, o1_ref, o11_ref):
    idx = idx_ref[...]
    def gath(comps_ref, out_ref):
        rb = out_ref.shape[0]
        comps = comps_ref[...]
        x3 = jnp.broadcast_to(comps[:, None, :], (rb, 128, 64))
        idx3 = jnp.broadcast_to(idx[None], (rb, 128, 128))
        out_ref[...] = jnp.take_along_axis(x3, idx3, axis=2)
    gath(k1_ref, o1_ref)
    gath(kv1_ref, o11_ref)


def _expand_b_body(oh32_ref, oh16_ref, k2_ref, k3_ref, kv2_ref, kv3_ref,
                   o2_ref, o3_ref, o22_ref, o33_ref):
    _expand(oh32_ref, k2_ref, o2_ref)
    _expand(oh16_ref, k3_ref, o3_ref)
    _expand(oh32_ref, kv2_ref, o22_ref)
    _expand(oh16_ref, kv3_ref, o33_ref)


def _full_spec(rows, cols):
    return pl.BlockSpec((rows, cols), lambda i: (0, 0))


def kernel(x, W1, b1, g1, beta1, W2, b2, g2, beta2, W3, b3, g3, beta3):
    f32 = jnp.float32
    x = x.astype(f32)
    fb = _freq_bands()
    w1c = W1[:_F, :]
    w1s = W1[_F:, :]
    row = lambda v: v.reshape(1, -1)

    kv, kvb = pl.pallas_call(
        _mlp_body,
        grid=(_G_MLP,),
        in_specs=[
            pl.BlockSpec((_B // _G_MLP, 1), lambda i: (i, 0)),
            _full_spec(1, _F),
            _full_spec(_F, _H1), _full_spec(_F, _H1),
            _full_spec(1, _H1), _full_spec(1, _H1), _full_spec(1, _H1),
            _full_spec(_H1, _D1),
            _full_spec(1, _D1), _full_spec(1, _D1), _full_spec(1, _D1),
            _full_spec(_D1, _D2),
            _full_spec(1, _D2), _full_spec(1, _D2), _full_spec(1, _D2),
        ],
        out_specs=[
            pl.BlockSpec((_B // _G_MLP, _D1), lambda i: (i, 0)),
            pl.BlockSpec((_B // _G_MLP, _D2), lambda i: (i, 0)),
        ],
        out_shape=[
            jax.ShapeDtypeStruct((_B, _D1), f32),
            jax.ShapeDtypeStruct((_B, _D2), f32),
        ],
        compiler_params=pltpu.CompilerParams(
            dimension_semantics=("parallel",),
        ),
        name="distance_mlp",
    )(x, fb, w1c, w1s, row(b1), row(g1), row(beta1),
      W2, row(b2), row(g2), row(beta2),
      W3, row(b3), row(g3), row(beta3))

    def comps_t(mat, n_ch, length):
        return (mat.reshape(_B, n_ch, length)
                   .transpose(2, 1, 0)
                   .reshape(length, n_ch * _B))

    kv2 = comps_t(kv[:, 256:512], 8, 32)
    kv3 = comps_t(kv[:, 512:], 16, 16)
    k2 = comps_t(kvb[:, 512:1024], 16, 32)
    k3 = comps_t(kvb[:, 1024:], 32, 16)
    kv1r = kv[:, :256].reshape(_B * 4, 64)
    k1r = kvb[:, :512].reshape(_B * 8, 64)

    g = _G_EXP
    r32, r16 = 4096 // g, 1024 // g
    o1, o11 = pl.pallas_call(
        _expand_a_body,
        grid=(g,),
        in_specs=[
            _full_spec(128, 128),
            pl.BlockSpec((_B * 8 // g, 64), lambda i: (i, 0)),
            pl.BlockSpec((_B * 4 // g, 64), lambda i: (i, 0)),
        ],
        out_specs=[
            pl.BlockSpec((_B * 8 // g, 128, 128), lambda i: (i, 0, 0)),
            pl.BlockSpec((_B * 4 // g, 128, 128), lambda i: (i, 0, 0)),
        ],
        out_shape=[
            jax.ShapeDtypeStruct((_B * 8, 128, 128), f32),
            jax.ShapeDtypeStruct((_B * 4, 128, 128), f32),
        ],
        compiler_params=pltpu.CompilerParams(
            dimension_semantics=("parallel",),
            vmem_limit_bytes=48 * 1024 * 1024,
        ),
        name="distance_expand_a",
    )(_bin_idx(64), k1r, kv1r)

    o2, o3, o22, o33 = pl.pallas_call(
        _expand_b_body,
        grid=(g,),
        in_specs=[
            pl.BlockSpec((r32, 32), lambda i: (i, 0)),
            pl.BlockSpec((r16, 16), lambda i: (i, 0)),
            _full_spec(32, 16 * _B),
            _full_spec(16, 32 * _B),
            _full_spec(32, 8 * _B),
            _full_spec(16, 16 * _B),
        ],
        out_specs=[
            pl.BlockSpec((16, 64 // g, 64, _B), lambda i: (0, i, 0, 0)),
            pl.BlockSpec((32, 32 // g, 32, _B), lambda i: (0, i, 0, 0)),
            pl.BlockSpec((8, 64 // g, 64, _B), lambda i: (0, i, 0, 0)),
            pl.BlockSpec((16, 32 // g, 32, _B), lambda i: (0, i, 0, 0)),
        ],
        out_shape=[
            jax.ShapeDtypeStruct((16, 64, 64, _B), f32),
            jax.ShapeDtypeStruct((32, 32, 32, _B), f32),
            jax.ShapeDtypeStruct((8, 64, 64, _B), f32),
            jax.ShapeDtypeStruct((16, 32, 32, _B), f32),
        ],
        compiler_params=pltpu.CompilerParams(
            dimension_semantics=("parallel",),
            vmem_limit_bytes=48 * 1024 * 1024,
        ),
        name="distance_expand_b",
    )(_bin_onehot_t(32), _bin_onehot_t(16), k2, k3, kv2, kv3)

    def finalize(o):
        return o.transpose(3, 0, 1, 2)

    c1 = o1.reshape(_B, 8, 128, 128)
    c2 = finalize(o2)
    c3 = finalize(o3)
    c11 = o11.reshape(_B, 4, 128, 128)
    c22 = finalize(o22)
    c33 = finalize(o33)
    return (c1, c2, c3, c11, c22, c33)


# speedup vs baseline: 5.5453x; 1.1073x over previous
import jax
import jax.numpy as jnp
from jax.experimental import pallas as pl
from jax.experimental.pallas import tpu as pltpu

_B = 128
_F = 32
_WL = 0.638
_PITCH = 8e-06
_H1 = 512
_D1 = 768
_D2 = 1536

_G_MLP = 2
_G_EXP = 8


def _freq_bands():
    wavelength = _WL * 1e-06
    min_fre = 2 * jnp.pi / wavelength * (1 - 2 * (wavelength / _PITCH / 2) ** 2) ** 0.5
    max_fre = 2 * jnp.pi / wavelength
    fb = (max_fre - min_fre) / _F * jnp.linspace(1.0, _F, _F) + min_fre
    return fb.astype(jnp.float32).reshape(1, _F)


def _bin_idx(length):
    ax = jnp.linspace(-float(length), float(length), 2 * length)
    xg, yg = jnp.meshgrid(ax, ax, indexing='ij')
    dis = jnp.sqrt(xg ** 2 + yg ** 2)
    interval = jnp.max(dis) / length
    return jnp.floor(dis / (interval + 0.0001)).astype(jnp.int32)


def _bin_onehot_t(length):
    idx = _bin_idx(length)
    flat = idx.reshape(4 * length * length, 1)
    lanes = jnp.arange(length, dtype=jnp.int32).reshape(1, length)
    return (flat == lanes).astype(jnp.float32)


def _ln_scale(z, g, b):
    m = jnp.mean(z, axis=-1, keepdims=True)
    v = jnp.mean((z - m) ** 2, axis=-1, keepdims=True)
    return (z - m) * jax.lax.rsqrt(v + 1e-5) * g + b


def _mlp_body(x_ref, fb_ref, w1c_ref, w1s_ref, b1_ref, g1_ref, be1_ref,
              w2_ref, b2_ref, g2_ref, be2_ref,
              w3_ref, b3_ref, g3_ref, be3_ref, kv_ref, kvb_ref):
    ds = x_ref[...] * fb_ref[...]
    z1 = (jnp.dot(jnp.cos(ds), w1c_ref[...], preferred_element_type=jnp.float32)
          + jnp.dot(jnp.sin(ds), w1s_ref[...], preferred_element_type=jnp.float32)
          + b1_ref[...])
    h = jnp.tanh(_ln_scale(z1, g1_ref[...], be1_ref[...]))
    z2 = jnp.dot(h, w2_ref[...], preferred_element_type=jnp.float32) + b2_ref[...]
    kv = jnp.tanh(_ln_scale(z2, g2_ref[...], be2_ref[...]))
    kv_ref[...] = kv
    z3 = jnp.dot(kv, w3_ref[...], preferred_element_type=jnp.float32) + b3_ref[...]
    kvb_ref[...] = jnp.tanh(_ln_scale(z3, g3_ref[...], be3_ref[...]))


def _expand(oh_ref, ct_ref, out_ref):
    n_ch, di = out_ref.shape[0], out_ref.shape[1]
    t = jnp.dot(oh_ref[...], ct_ref[...],
                preferred_element_type=jnp.float32)
    for c in range(n_ch):
        out_ref[c] = t[:, c * _B:(c + 1) * _B].reshape(di, -1, _B)


def _expand_all_body(idx_ref, oh32_ref, oh16_ref,
                     k1_ref, k2_ref, k3_ref, kv1_ref, kv2_ref, kv3_ref,
                     o1_ref, o2_ref, o3_ref, o11_ref, o22_ref, o33_ref):
    idx = idx_ref[...]
    def gath(comps_ref, out_ref):
        rb = out_ref.shape[0]
        comps = comps_ref[...]
        x3 = jnp.broadcast_to(comps[:, None, :], (rb, 128, 64))
        idx3 = jnp.broadcast_to(idx[None], (rb, 128, 128))
        out_ref[...] = jnp.take_along_axis(x3, idx3, axis=2)
    gath(k1_ref, o1_ref)
    gath(kv1_ref, o11_ref)
    _expand(oh32_ref, k2_ref, o2_ref)
    _expand(oh16_ref, k3_ref, o3_ref)
    _expand(oh32_ref, kv2_ref, o22_ref)
    _expand(oh16_ref, kv3_ref, o33_ref)


def _full_spec(rows, cols):
    return pl.BlockSpec((rows, cols), lambda i: (0, 0))


def kernel(x, W1, b1, g1, beta1, W2, b2, g2, beta2, W3, b3, g3, beta3):
    f32 = jnp.float32
    x = x.astype(f32)
    fb = _freq_bands()
    w1c = W1[:_F, :]
    w1s = W1[_F:, :]
    row = lambda v: v.reshape(1, -1)

    kv, kvb = pl.pallas_call(
        _mlp_body,
        grid=(_G_MLP,),
        in_specs=[
            pl.BlockSpec((_B // _G_MLP, 1), lambda i: (i, 0)),
            _full_spec(1, _F),
            _full_spec(_F, _H1), _full_spec(_F, _H1),
            _full_spec(1, _H1), _full_spec(1, _H1), _full_spec(1, _H1),
            _full_spec(_H1, _D1),
            _full_spec(1, _D1), _full_spec(1, _D1), _full_spec(1, _D1),
            _full_spec(_D1, _D2),
            _full_spec(1, _D2), _full_spec(1, _D2), _full_spec(1, _D2),
        ],
        out_specs=[
            pl.BlockSpec((_B // _G_MLP, _D1), lambda i: (i, 0)),
            pl.BlockSpec((_B // _G_MLP, _D2), lambda i: (i, 0)),
        ],
        out_shape=[
            jax.ShapeDtypeStruct((_B, _D1), f32),
            jax.ShapeDtypeStruct((_B, _D2), f32),
        ],
        compiler_params=pltpu.CompilerParams(
            dimension_semantics=("parallel",),
        ),
        name="distance_mlp",
    )(x, fb, w1c, w1s, row(b1), row(g1), row(beta1),
      W2, row(b2), row(g2), row(beta2),
      W3, row(b3), row(g3), row(beta3))

    def comps_t(mat, n_ch, length):
        return (mat.reshape(_B, n_ch, length)
                   .transpose(2, 1, 0)
                   .reshape(length, n_ch * _B))

    kv2 = comps_t(kv[:, 256:512], 8, 32)
    kv3 = comps_t(kv[:, 512:], 16, 16)
    k2 = comps_t(kvb[:, 512:1024], 16, 32)
    k3 = comps_t(kvb[:, 1024:], 32, 16)
    kv1r = kv[:, :256].reshape(_B * 4, 64)
    k1r = kvb[:, :512].reshape(_B * 8, 64)

    g = _G_EXP
    r32, r16 = 4096 // g, 1024 // g
    o1, o2, o3, o11, o22, o33 = pl.pallas_call(
        _expand_all_body,
        grid=(g,),
        in_specs=[
            _full_spec(128, 128),
            pl.BlockSpec((r32, 32), lambda i: (i, 0)),
            pl.BlockSpec((r16, 16), lambda i: (i, 0)),
            pl.BlockSpec((_B * 8 // g, 64), lambda i: (i, 0)),
            _full_spec(32, 16 * _B),
            _full_spec(16, 32 * _B),
            pl.BlockSpec((_B * 4 // g, 64), lambda i: (i, 0)),
            _full_spec(32, 8 * _B),
            _full_spec(16, 16 * _B),
        ],
        out_specs=[
            pl.BlockSpec((_B * 8 // g, 128, 128), lambda i: (i, 0, 0)),
            pl.BlockSpec((16, 64 // g, 64, _B), lambda i: (0, i, 0, 0)),
            pl.BlockSpec((32, 32 // g, 32, _B), lambda i: (0, i, 0, 0)),
            pl.BlockSpec((_B * 4 // g, 128, 128), lambda i: (i, 0, 0)),
            pl.BlockSpec((8, 64 // g, 64, _B), lambda i: (0, i, 0, 0)),
            pl.BlockSpec((16, 32 // g, 32, _B), lambda i: (0, i, 0, 0)),
        ],
        out_shape=[
            jax.ShapeDtypeStruct((_B * 8, 128, 128), f32),
            jax.ShapeDtypeStruct((16, 64, 64, _B), f32),
            jax.ShapeDtypeStruct((32, 32, 32, _B), f32),
            jax.ShapeDtypeStruct((_B * 4, 128, 128), f32),
            jax.ShapeDtypeStruct((8, 64, 64, _B), f32),
            jax.ShapeDtypeStruct((16, 32, 32, _B), f32),
        ],
        compiler_params=pltpu.CompilerParams(
            dimension_semantics=("parallel",),
            vmem_limit_bytes=56 * 1024 * 1024,
        ),
        name="distance_expand",
    )(_bin_idx(64), _bin_onehot_t(32), _bin_onehot_t(16),
      k1r, k2, k3, kv1r, kv2, kv3)

    def finalize(o):
        return o.transpose(3, 0, 1, 2)

    c1 = o1.reshape(_B, 8, 128, 128)
    c2 = finalize(o2)
    c3 = finalize(o3)
    c11 = o11.reshape(_B, 4, 128, 128)
    c22 = finalize(o22)
    c33 = finalize(o33)
    return (c1, c2, c3, c11, c22, c33)


# speedup vs baseline: 6.6913x; 1.2067x over previous
import jax
import jax.numpy as jnp
from jax.experimental import pallas as pl
from jax.experimental.pallas import tpu as pltpu

_B = 128
_F = 32
_WL = 0.638
_PITCH = 8e-06
_H1 = 512
_D1 = 768
_D2 = 1536

_G_MLP = 2
_G_EXP = 8


def _freq_bands():
    wavelength = _WL * 1e-06
    min_fre = 2 * jnp.pi / wavelength * (1 - 2 * (wavelength / _PITCH / 2) ** 2) ** 0.5
    max_fre = 2 * jnp.pi / wavelength
    fb = (max_fre - min_fre) / _F * jnp.linspace(1.0, _F, _F) + min_fre
    return fb.astype(jnp.float32).reshape(1, _F)


def _bin_idx(length):
    ax = jnp.linspace(-float(length), float(length), 2 * length)
    xg, yg = jnp.meshgrid(ax, ax, indexing='ij')
    dis = jnp.sqrt(xg ** 2 + yg ** 2)
    interval = jnp.max(dis) / length
    return jnp.floor(dis / (interval + 0.0001)).astype(jnp.int32)


def _bin_onehot_t(length):
    idx = _bin_idx(length)
    flat = idx.reshape(4 * length * length, 1)
    lanes = jnp.arange(length, dtype=jnp.int32).reshape(1, length)
    return (flat == lanes).astype(jnp.float32)


def _ln_scale(z, g, b):
    m = jnp.mean(z, axis=-1, keepdims=True)
    v = jnp.mean((z - m) ** 2, axis=-1, keepdims=True)
    return (z - m) * jax.lax.rsqrt(v + 1e-5) * g + b


def _mlp_body(x_ref, fb_ref, w1c_ref, w1s_ref, b1_ref, g1_ref, be1_ref,
              w2_ref, b2_ref, g2_ref, be2_ref,
              w3_ref, b3_ref, g3_ref, be3_ref, kv_ref, kvb_ref):
    ds = x_ref[...] * fb_ref[...]
    z1 = (jnp.dot(jnp.cos(ds), w1c_ref[...], preferred_element_type=jnp.float32)
          + jnp.dot(jnp.sin(ds), w1s_ref[...], preferred_element_type=jnp.float32)
          + b1_ref[...])
    h = jnp.tanh(_ln_scale(z1, g1_ref[...], be1_ref[...]))
    z2 = jnp.dot(h, w2_ref[...], preferred_element_type=jnp.float32) + b2_ref[...]
    kv = jnp.tanh(_ln_scale(z2, g2_ref[...], be2_ref[...]))
    kv_ref[...] = kv
    z3 = jnp.dot(kv, w3_ref[...], preferred_element_type=jnp.float32) + b3_ref[...]
    kvb_ref[...] = jnp.tanh(_ln_scale(z3, g3_ref[...], be3_ref[...]))


def _expand(oh_ref, ct_ref, out_ref):
    n_ch, di = out_ref.shape[0], out_ref.shape[1]
    t = jnp.dot(oh_ref[...], ct_ref[...],
                preferred_element_type=jnp.float32)
    for c in range(n_ch):
        out_ref[c] = t[:, c * _B:(c + 1) * _B].reshape(di, -1, _B)


def _expand_all_body(idx_ref, oh32_ref, oh16_ref,
                     k1_ref, k2_ref, k3_ref, kv1_ref, kv2_ref, kv3_ref,
                     o1_ref, o2_ref, o3_ref, o11_ref, o22_ref, o33_ref):
    idx = idx_ref[...]
    def gath(comps_ref, out_ref):
        rb = out_ref.shape[0]
        comps = comps_ref[...]
        x3 = jnp.broadcast_to(comps[:, None, :], (rb, 8, 64))
        for o in range(16):
            idx3 = jnp.broadcast_to(idx[None, 8 * o:8 * (o + 1), :],
                                    (rb, 8, 128))
            out_ref[:, 8 * o:8 * (o + 1), :] = (
                jnp.take_along_axis(x3, idx3, axis=2))
    gath(k1_ref, o1_ref)
    gath(kv1_ref, o11_ref)
    _expand(oh32_ref, k2_ref, o2_ref)
    _expand(oh16_ref, k3_ref, o3_ref)
    _expand(oh32_ref, kv2_ref, o22_ref)
    _expand(oh16_ref, kv3_ref, o33_ref)


def _full_spec(rows, cols):
    return pl.BlockSpec((rows, cols), lambda i: (0, 0))


def kernel(x, W1, b1, g1, beta1, W2, b2, g2, beta2, W3, b3, g3, beta3):
    f32 = jnp.float32
    x = x.astype(f32)
    fb = _freq_bands()
    w1c = W1[:_F, :]
    w1s = W1[_F:, :]
    row = lambda v: v.reshape(1, -1)

    kv, kvb = pl.pallas_call(
        _mlp_body,
        grid=(_G_MLP,),
        in_specs=[
            pl.BlockSpec((_B // _G_MLP, 1), lambda i: (i, 0)),
            _full_spec(1, _F),
            _full_spec(_F, _H1), _full_spec(_F, _H1),
            _full_spec(1, _H1), _full_spec(1, _H1), _full_spec(1, _H1),
            _full_spec(_H1, _D1),
            _full_spec(1, _D1), _full_spec(1, _D1), _full_spec(1, _D1),
            _full_spec(_D1, _D2),
            _full_spec(1, _D2), _full_spec(1, _D2), _full_spec(1, _D2),
        ],
        out_specs=[
            pl.BlockSpec((_B // _G_MLP, _D1), lambda i: (i, 0)),
            pl.BlockSpec((_B // _G_MLP, _D2), lambda i: (i, 0)),
        ],
        out_shape=[
            jax.ShapeDtypeStruct((_B, _D1), f32),
            jax.ShapeDtypeStruct((_B, _D2), f32),
        ],
        compiler_params=pltpu.CompilerParams(
            dimension_semantics=("parallel",),
        ),
        name="distance_mlp",
    )(x, fb, w1c, w1s, row(b1), row(g1), row(beta1),
      W2, row(b2), row(g2), row(beta2),
      W3, row(b3), row(g3), row(beta3))

    def comps_t(mat, n_ch, length):
        return (mat.reshape(_B, n_ch, length)
                   .transpose(2, 1, 0)
                   .reshape(length, n_ch * _B))

    kv2 = comps_t(kv[:, 256:512], 8, 32)
    kv3 = comps_t(kv[:, 512:], 16, 16)
    k2 = comps_t(kvb[:, 512:1024], 16, 32)
    k3 = comps_t(kvb[:, 1024:], 32, 16)
    kv1r = kv[:, :256].reshape(_B * 4, 64)
    k1r = kvb[:, :512].reshape(_B * 8, 64)

    g = _G_EXP
    r32, r16 = 4096 // g, 1024 // g
    o1, o2, o3, o11, o22, o33 = pl.pallas_call(
        _expand_all_body,
        grid=(g,),
        in_specs=[
            _full_spec(128, 128),
            pl.BlockSpec((r32, 32), lambda i: (i, 0)),
            pl.BlockSpec((r16, 16), lambda i: (i, 0)),
            pl.BlockSpec((_B * 8 // g, 64), lambda i: (i, 0)),
            _full_spec(32, 16 * _B),
            _full_spec(16, 32 * _B),
            pl.BlockSpec((_B * 4 // g, 64), lambda i: (i, 0)),
            _full_spec(32, 8 * _B),
            _full_spec(16, 16 * _B),
        ],
        out_specs=[
            pl.BlockSpec((_B * 8 // g, 128, 128), lambda i: (i, 0, 0)),
            pl.BlockSpec((16, 64 // g, 64, _B), lambda i: (0, i, 0, 0)),
            pl.BlockSpec((32, 32 // g, 32, _B), lambda i: (0, i, 0, 0)),
            pl.BlockSpec((_B * 4 // g, 128, 128), lambda i: (i, 0, 0)),
            pl.BlockSpec((8, 64 // g, 64, _B), lambda i: (0, i, 0, 0)),
            pl.BlockSpec((16, 32 // g, 32, _B), lambda i: (0, i, 0, 0)),
        ],
        out_shape=[
            jax.ShapeDtypeStruct((_B * 8, 128, 128), f32),
            jax.ShapeDtypeStruct((16, 64, 64, _B), f32),
            jax.ShapeDtypeStruct((32, 32, 32, _B), f32),
            jax.ShapeDtypeStruct((_B * 4, 128, 128), f32),
            jax.ShapeDtypeStruct((8, 64, 64, _B), f32),
            jax.ShapeDtypeStruct((16, 32, 32, _B), f32),
        ],
        compiler_params=pltpu.CompilerParams(
            dimension_semantics=("parallel",),
            vmem_limit_bytes=56 * 1024 * 1024,
        ),
        name="distance_expand",
    )(_bin_idx(64), _bin_onehot_t(32), _bin_onehot_t(16),
      k1r, k2, k3, kv1r, kv2, kv3)

    def finalize(o):
        return o.transpose(3, 0, 1, 2)

    c1 = o1.reshape(_B, 8, 128, 128)
    c2 = finalize(o2)
    c3 = finalize(o3)
    c11 = o11.reshape(_B, 4, 128, 128)
    c22 = finalize(o22)
    c33 = finalize(o33)
    return (c1, c2, c3, c11, c22, c33)


# speedup vs baseline: 7.3050x; 1.0917x over previous
import jax
import jax.numpy as jnp
from jax.experimental import pallas as pl
from jax.experimental.pallas import tpu as pltpu

_B = 128
_F = 32
_WL = 0.638
_PITCH = 8e-06
_H1 = 512
_D1 = 768
_D2 = 1536

_G_MLP = 2
_G_EXP = 8


def _freq_bands():
    wavelength = _WL * 1e-06
    min_fre = 2 * jnp.pi / wavelength * (1 - 2 * (wavelength / _PITCH / 2) ** 2) ** 0.5
    max_fre = 2 * jnp.pi / wavelength
    fb = (max_fre - min_fre) / _F * jnp.linspace(1.0, _F, _F) + min_fre
    return fb.astype(jnp.float32).reshape(1, _F)


def _bin_idx(length):
    ax = jnp.linspace(-float(length), float(length), 2 * length)
    xg, yg = jnp.meshgrid(ax, ax, indexing='ij')
    dis = jnp.sqrt(xg ** 2 + yg ** 2)
    interval = jnp.max(dis) / length
    return jnp.floor(dis / (interval + 0.0001)).astype(jnp.int32)


def _bin_onehot_t(length):
    idx = _bin_idx(length)
    flat = idx.reshape(4 * length * length, 1)
    lanes = jnp.arange(length, dtype=jnp.int32).reshape(1, length)
    return (flat == lanes).astype(jnp.float32)


def _ln_scale(z, g, b):
    m = jnp.mean(z, axis=-1, keepdims=True)
    v = jnp.mean((z - m) ** 2, axis=-1, keepdims=True)
    return (z - m) * jax.lax.rsqrt(v + 1e-5) * g + b


def _mlp_body(x_ref, fb_ref, w1c_ref, w1s_ref, b1_ref, g1_ref, be1_ref,
              w2_ref, b2_ref, g2_ref, be2_ref,
              w3_ref, b3_ref, g3_ref, be3_ref, kv_ref, kvb_ref):
    ds = x_ref[...] * fb_ref[...]
    z1 = (jnp.dot(jnp.cos(ds), w1c_ref[...], preferred_element_type=jnp.float32)
          + jnp.dot(jnp.sin(ds), w1s_ref[...], preferred_element_type=jnp.float32)
          + b1_ref[...])
    h = jnp.tanh(_ln_scale(z1, g1_ref[...], be1_ref[...]))
    z2 = jnp.dot(h, w2_ref[...], preferred_element_type=jnp.float32) + b2_ref[...]
    kv = jnp.tanh(_ln_scale(z2, g2_ref[...], be2_ref[...]))
    kv_ref[...] = kv
    z3 = jnp.dot(kv, w3_ref[...], preferred_element_type=jnp.float32) + b3_ref[...]
    kvb_ref[...] = jnp.tanh(_ln_scale(z3, g3_ref[...], be3_ref[...]))


def _expand(oh_ref, ct_ref, out_ref):
    n_ch, di = out_ref.shape[0], out_ref.shape[1]
    t = jnp.dot(oh_ref[...], ct_ref[...],
                preferred_element_type=jnp.float32)
    for c in range(n_ch):
        out_ref[c] = t[:, c * _B:(c + 1) * _B].reshape(di, -1, _B)


def _expand_all_body(idx_ref, oh32_ref, oh16_ref,
                     k1_ref, k2_ref, k3_ref, kv1_ref, kv2_ref, kv3_ref,
                     o1_ref, o2_ref, o3_ref, o11_ref, o22_ref, o33_ref):
    idx = idx_ref[...]
    def gath(comps_ref, out_ref):
        rb = out_ref.shape[0]
        comps = comps_ref[...]
        x3 = jnp.broadcast_to(comps[:, None, :], (rb, 8, 64))
        for o in range(16):
            idx3 = jnp.broadcast_to(idx[None, 8 * o:8 * (o + 1), :],
                                    (rb, 8, 128))
            out_ref[:, 8 * o:8 * (o + 1), :] = (
                jnp.take_along_axis(x3, idx3, axis=2))
    gath(k1_ref, o1_ref)
    gath(kv1_ref, o11_ref)
    _expand(oh32_ref, k2_ref, o2_ref)
    _expand(oh16_ref, k3_ref, o3_ref)
    _expand(oh32_ref, kv2_ref, o22_ref)
    _expand(oh16_ref, kv3_ref, o33_ref)


def _full_spec(rows, cols):
    return pl.BlockSpec((rows, cols), lambda i: (0, 0))


def kernel(x, W1, b1, g1, beta1, W2, b2, g2, beta2, W3, b3, g3, beta3):
    f32 = jnp.float32
    x = x.astype(f32)
    fb = _freq_bands()
    w1c = W1[:_F, :]
    w1s = W1[_F:, :]

    def vec_spec(n):
        return pl.BlockSpec((n,), lambda i: (0,))

    kv, kvb = pl.pallas_call(
        _mlp_body,
        grid=(_G_MLP,),
        in_specs=[
            pl.BlockSpec((_B // _G_MLP, 1), lambda i: (i, 0)),
            _full_spec(1, _F),
            _full_spec(_F, _H1), _full_spec(_F, _H1),
            vec_spec(_H1), vec_spec(_H1), vec_spec(_H1),
            _full_spec(_H1, _D1),
            vec_spec(_D1), vec_spec(_D1), vec_spec(_D1),
            _full_spec(_D1, _D2),
            vec_spec(_D2), vec_spec(_D2), vec_spec(_D2),
        ],
        out_specs=[
            pl.BlockSpec((_B // _G_MLP, _D1), lambda i: (i, 0)),
            pl.BlockSpec((_B // _G_MLP, _D2), lambda i: (i, 0)),
        ],
        out_shape=[
            jax.ShapeDtypeStruct((_B, _D1), f32),
            jax.ShapeDtypeStruct((_B, _D2), f32),
        ],
        compiler_params=pltpu.CompilerParams(
            dimension_semantics=("parallel",),
        ),
        name="distance_mlp",
    )(x, fb, w1c, w1s, b1, g1, beta1,
      W2, b2, g2, beta2,
      W3, b3, g3, beta3)

    def comps_t(mat, n_ch, length):
        return (mat.reshape(_B, n_ch, length)
                   .transpose(2, 1, 0)
                   .reshape(length, n_ch * _B))

    kv2 = comps_t(kv[:, 256:512], 8, 32)
    kv3 = comps_t(kv[:, 512:], 16, 16)
    k2 = comps_t(kvb[:, 512:1024], 16, 32)
    k3 = comps_t(kvb[:, 1024:], 32, 16)
    kv1r = kv[:, :256].reshape(_B * 4, 64)
    k1r = kvb[:, :512].reshape(_B * 8, 64)

    g = _G_EXP
    r32, r16 = 4096 // g, 1024 // g
    o1, o2, o3, o11, o22, o33 = pl.pallas_call(
        _expand_all_body,
        grid=(g,),
        in_specs=[
            _full_spec(128, 128),
            pl.BlockSpec((r32, 32), lambda i: (i, 0)),
            pl.BlockSpec((r16, 16), lambda i: (i, 0)),
            pl.BlockSpec((_B * 8 // g, 64), lambda i: (i, 0)),
            _full_spec(32, 16 * _B),
            _full_spec(16, 32 * _B),
            pl.BlockSpec((_B * 4 // g, 64), lambda i: (i, 0)),
            _full_spec(32, 8 * _B),
            _full_spec(16, 16 * _B),
        ],
        out_specs=[
            pl.BlockSpec((_B * 8 // g, 128, 128), lambda i: (i, 0, 0)),
            pl.BlockSpec((16, 64 // g, 64, _B), lambda i: (0, i, 0, 0)),
            pl.BlockSpec((32, 32 // g, 32, _B), lambda i: (0, i, 0, 0)),
            pl.BlockSpec((_B * 4 // g, 128, 128), lambda i: (i, 0, 0)),
            pl.BlockSpec((8, 64 // g, 64, _B), lambda i: (0, i, 0, 0)),
            pl.BlockSpec((16, 32 // g, 32, _B), lambda i: (0, i, 0, 0)),
        ],
        out_shape=[
            jax.ShapeDtypeStruct((_B * 8, 128, 128), f32),
            jax.ShapeDtypeStruct((16, 64, 64, _B), f32),
            jax.ShapeDtypeStruct((32, 32, 32, _B), f32),
            jax.ShapeDtypeStruct((_B * 4, 128, 128), f32),
            jax.ShapeDtypeStruct((8, 64, 64, _B), f32),
            jax.ShapeDtypeStruct((16, 32, 32, _B), f32),
        ],
        compiler_params=pltpu.CompilerParams(
            dimension_semantics=("parallel",),
            vmem_limit_bytes=56 * 1024 * 1024,
        ),
        name="distance_expand",
    )(_bin_idx(64), _bin_onehot_t(32), _bin_onehot_t(16),
      k1r, k2, k3, kv1r, kv2, kv3)

    def finalize(o):
        return o.transpose(3, 0, 1, 2)

    c1 = o1.reshape(_B, 8, 128, 128)
    c2 = finalize(o2)
    c3 = finalize(o3)
    c11 = o11.reshape(_B, 4, 128, 128)
    c22 = finalize(o22)
    c33 = finalize(o33)
    return (c1, c2, c3, c11, c22, c33)


# speedup vs baseline: 7.3327x; 1.0038x over previous
import jax
import jax.numpy as jnp
from jax.experimental import pallas as pl
from jax.experimental.pallas import tpu as pltpu

_B = 128
_F = 32
_WL = 0.638
_PITCH = 8e-06
_H1 = 512
_D1 = 768
_D2 = 1536

_G_MLP = 2
_G_EXP = 8


import numpy as np

_WAVELENGTH = _WL * 1e-06
_MIN_FRE = 2 * np.pi / _WAVELENGTH * (1 - 2 * (_WAVELENGTH / _PITCH / 2) ** 2) ** 0.5
_MAX_FRE = 2 * np.pi / _WAVELENGTH
_FB_SCALE = np.float32((_MAX_FRE - _MIN_FRE) / _F)
_FB_MIN = np.float32(_MIN_FRE)


def _bin_idx(length):
    ax = jnp.linspace(-float(length), float(length), 2 * length)
    xg, yg = jnp.meshgrid(ax, ax, indexing='ij')
    dis = jnp.sqrt(xg ** 2 + yg ** 2)
    interval = jnp.max(dis) / length
    return jnp.floor(dis / (interval + 0.0001)).astype(jnp.int32)


def _bin_onehot_t(length):
    idx = _bin_idx(length)
    flat = idx.reshape(4 * length * length, 1)
    lanes = jnp.arange(length, dtype=jnp.int32).reshape(1, length)
    return (flat == lanes).astype(jnp.float32)


def _ln_scale(z, g, b):
    m = jnp.mean(z, axis=-1, keepdims=True)
    v = jnp.mean((z - m) ** 2, axis=-1, keepdims=True)
    return (z - m) * jax.lax.rsqrt(v + 1e-5) * g + b


def _mlp_body(x_ref, w1c_ref, w1s_ref, b1_ref, g1_ref, be1_ref,
              w2_ref, b2_ref, g2_ref, be2_ref,
              w3_ref, b3_ref, g3_ref, be3_ref, kv_ref, kvb_ref):
    bands = (jax.lax.broadcasted_iota(jnp.int32, (1, _F), 1) + 1).astype(jnp.float32)
    fb = bands * _FB_SCALE + _FB_MIN
    ds = x_ref[...] * fb
    z1 = (jnp.dot(jnp.cos(ds), w1c_ref[...], preferred_element_type=jnp.float32)
          + jnp.dot(jnp.sin(ds), w1s_ref[...], preferred_element_type=jnp.float32)
          + b1_ref[...])
    h = jnp.tanh(_ln_scale(z1, g1_ref[...], be1_ref[...]))
    z2 = jnp.dot(h, w2_ref[...], preferred_element_type=jnp.float32) + b2_ref[...]
    kv = jnp.tanh(_ln_scale(z2, g2_ref[...], be2_ref[...]))
    kv_ref[...] = kv
    z3 = jnp.dot(kv, w3_ref[...], preferred_element_type=jnp.float32) + b3_ref[...]
    kvb_ref[...] = jnp.tanh(_ln_scale(z3, g3_ref[...], be3_ref[...]))


def _expand(oh_ref, ct_ref, out_ref):
    n_ch, di = out_ref.shape[0], out_ref.shape[1]
    t = jnp.dot(oh_ref[...], ct_ref[...],
                preferred_element_type=jnp.float32)
    for c in range(n_ch):
        out_ref[c] = t[:, c * _B:(c + 1) * _B].reshape(di, -1, _B)


def _expand_all_body(idx_ref, oh32_ref, oh16_ref,
                     k1_ref, k2_ref, k3_ref, kv1_ref, kv2_ref, kv3_ref,
                     o1_ref, o2_ref, o3_ref, o11_ref, o22_ref, o33_ref):
    idx = idx_ref[...]
    def gath(comps_ref, out_ref):
        rb = out_ref.shape[0]
        comps = comps_ref[...]
        x3 = jnp.broadcast_to(comps[:, None, :], (rb, 8, 64))
        for o in range(16):
            idx3 = jnp.broadcast_to(idx[None, 8 * o:8 * (o + 1), :],
                                    (rb, 8, 128))
            out_ref[:, 8 * o:8 * (o + 1), :] = (
                jnp.take_along_axis(x3, idx3, axis=2))
    gath(k1_ref, o1_ref)
    gath(kv1_ref, o11_ref)
    _expand(oh32_ref, k2_ref, o2_ref)
    _expand(oh16_ref, k3_ref, o3_ref)
    _expand(oh32_ref, kv2_ref, o22_ref)
    _expand(oh16_ref, kv3_ref, o33_ref)


def _full_spec(rows, cols):
    return pl.BlockSpec((rows, cols), lambda i: (0, 0))


def kernel(x, W1, b1, g1, beta1, W2, b2, g2, beta2, W3, b3, g3, beta3):
    f32 = jnp.float32
    w1c = W1[:_F, :]
    w1s = W1[_F:, :]

    def vec_spec(n):
        return pl.BlockSpec((n,), lambda i: (0,))

    kv, kvb = pl.pallas_call(
        _mlp_body,
        grid=(_G_MLP,),
        in_specs=[
            pl.BlockSpec((_B // _G_MLP, 1), lambda i: (i, 0)),
            _full_spec(_F, _H1), _full_spec(_F, _H1),
            vec_spec(_H1), vec_spec(_H1), vec_spec(_H1),
            _full_spec(_H1, _D1),
            vec_spec(_D1), vec_spec(_D1), vec_spec(_D1),
            _full_spec(_D1, _D2),
            vec_spec(_D2), vec_spec(_D2), vec_spec(_D2),
        ],
        out_specs=[
            pl.BlockSpec((_B // _G_MLP, _D1), lambda i: (i, 0)),
            pl.BlockSpec((_B // _G_MLP, _D2), lambda i: (i, 0)),
        ],
        out_shape=[
            jax.ShapeDtypeStruct((_B, _D1), f32),
            jax.ShapeDtypeStruct((_B, _D2), f32),
        ],
        compiler_params=pltpu.CompilerParams(
            dimension_semantics=("parallel",),
        ),
        name="distance_mlp",
    )(x, w1c, w1s, b1, g1, beta1,
      W2, b2, g2, beta2,
      W3, b3, g3, beta3)

    def comps_t(mat, n_ch, length):
        return (mat.reshape(_B, n_ch, length)
                   .transpose(2, 1, 0)
                   .reshape(length, n_ch * _B))

    kv2 = comps_t(kv[:, 256:512], 8, 32)
    kv3 = comps_t(kv[:, 512:], 16, 16)
    k2 = comps_t(kvb[:, 512:1024], 16, 32)
    k3 = comps_t(kvb[:, 1024:], 32, 16)
    kv1r = kv[:, :256].reshape(_B * 4, 64)
    k1r = kvb[:, :512].reshape(_B * 8, 64)

    g = _G_EXP
    r32, r16 = 4096 // g, 1024 // g
    o1, o2, o3, o11, o22, o33 = pl.pallas_call(
        _expand_all_body,
        grid=(g,),
        in_specs=[
            _full_spec(128, 128),
            pl.BlockSpec((r32, 32), lambda i: (i, 0)),
            pl.BlockSpec((r16, 16), lambda i: (i, 0)),
            pl.BlockSpec((_B * 8 // g, 64), lambda i: (i, 0)),
            _full_spec(32, 16 * _B),
            _full_spec(16, 32 * _B),
            pl.BlockSpec((_B * 4 // g, 64), lambda i: (i, 0)),
            _full_spec(32, 8 * _B),
            _full_spec(16, 16 * _B),
        ],
        out_specs=[
            pl.BlockSpec((_B * 8 // g, 128, 128), lambda i: (i, 0, 0)),
            pl.BlockSpec((16, 64 // g, 64, _B), lambda i: (0, i, 0, 0)),
            pl.BlockSpec((32, 32 // g, 32, _B), lambda i: (0, i, 0, 0)),
            pl.BlockSpec((_B * 4 // g, 128, 128), lambda i: (i, 0, 0)),
            pl.BlockSpec((8, 64 // g, 64, _B), lambda i: (0, i, 0, 0)),
            pl.BlockSpec((16, 32 // g, 32, _B), lambda i: (0, i, 0, 0)),
        ],
        out_shape=[
            jax.ShapeDtypeStruct((_B * 8, 128, 128), f32),
            jax.ShapeDtypeStruct((16, 64, 64, _B), f32),
            jax.ShapeDtypeStruct((32, 32, 32, _B), f32),
            jax.ShapeDtypeStruct((_B * 4, 128, 128), f32),
            jax.ShapeDtypeStruct((8, 64, 64, _B), f32),
            jax.ShapeDtypeStruct((16, 32, 32, _B), f32),
        ],
        compiler_params=pltpu.CompilerParams(
            dimension_semantics=("parallel",),
            vmem_limit_bytes=56 * 1024 * 1024,
        ),
        name="distance_expand",
    )(_bin_idx(64), _bin_onehot_t(32), _bin_onehot_t(16),
      k1r, k2, k3, kv1r, kv2, kv3)

    def finalize(o):
        return o.transpose(3, 0, 1, 2)

    c1 = o1.reshape(_B, 8, 128, 128)
    c2 = finalize(o2)
    c3 = finalize(o3)
    c11 = o11.reshape(_B, 4, 128, 128)
    c22 = finalize(o22)
    c33 = finalize(o33)
    return (c1, c2, c3, c11, c22, c33)


# speedup vs baseline: 7.5759x; 1.0332x over previous
import jax
import jax.numpy as jnp
from jax.experimental import pallas as pl
from jax.experimental.pallas import tpu as pltpu

_B = 128
_F = 32
_WL = 0.638
_PITCH = 8e-06
_H1 = 512
_D1 = 768
_D2 = 1536

_G_MLP = 2
_G_EXP = 8


import numpy as np

_WAVELENGTH = _WL * 1e-06
_MIN_FRE = 2 * np.pi / _WAVELENGTH * (1 - 2 * (_WAVELENGTH / _PITCH / 2) ** 2) ** 0.5
_MAX_FRE = 2 * np.pi / _WAVELENGTH
_FB_SCALE = np.float32((_MAX_FRE - _MIN_FRE) / _F)
_FB_MIN = np.float32(_MIN_FRE)


def _bin_idx(length):
    ax = jnp.linspace(-float(length), float(length), 2 * length)
    xg, yg = jnp.meshgrid(ax, ax, indexing='ij')
    dis = jnp.sqrt(xg ** 2 + yg ** 2)
    interval = jnp.max(dis) / length
    return jnp.floor(dis / (interval + 0.0001)).astype(jnp.int32)


def _bin_onehot_t(length):
    idx = _bin_idx(length)
    flat = idx.reshape(4 * length * length, 1)
    lanes = jnp.arange(length, dtype=jnp.int32).reshape(1, length)
    return (flat == lanes).astype(jnp.float32)


def _ln_scale(z, g, b):
    m = jnp.mean(z, axis=-1, keepdims=True)
    v = jnp.mean((z - m) ** 2, axis=-1, keepdims=True)
    return (z - m) * jax.lax.rsqrt(v + 1e-5) * g + b


def _mlp_body(x_ref, w1_ref, b1_ref, g1_ref, be1_ref,
              w2_ref, b2_ref, g2_ref, be2_ref,
              w3_ref, b3_ref, g3_ref, be3_ref, kv_ref, kvb_ref):
    bands = (jax.lax.broadcasted_iota(jnp.int32, (1, _F), 1) + 1).astype(jnp.float32)
    fb = bands * _FB_SCALE + _FB_MIN
    ds = x_ref[...] * fb
    z1 = (jnp.dot(jnp.cos(ds), w1_ref[:_F, :], preferred_element_type=jnp.float32)
          + jnp.dot(jnp.sin(ds), w1_ref[_F:, :], preferred_element_type=jnp.float32)
          + b1_ref[...])
    h = jnp.tanh(_ln_scale(z1, g1_ref[...], be1_ref[...]))
    z2 = jnp.dot(h, w2_ref[...], preferred_element_type=jnp.float32) + b2_ref[...]
    kv = jnp.tanh(_ln_scale(z2, g2_ref[...], be2_ref[...]))
    kv_ref[...] = kv
    z3 = jnp.dot(kv, w3_ref[...], preferred_element_type=jnp.float32) + b3_ref[...]
    kvb_ref[...] = jnp.tanh(_ln_scale(z3, g3_ref[...], be3_ref[...]))


def _expand(oh_ref, ct, out_ref):
    n_ch, di = out_ref.shape[0], out_ref.shape[1]
    t = jnp.dot(oh_ref[...], ct,
                preferred_element_type=jnp.float32)
    for c in range(n_ch):
        out_ref[c] = t[:, c * _B:(c + 1) * _B].reshape(di, -1, _B)


def _expand_all_body(idx_ref, oh32_ref, oh16_ref,
                     k1_ref, ct32_ref, ct16_ref, kv1_ref,
                     o1_ref, o2_ref, o3_ref, o11_ref, o22_ref, o33_ref):
    idx = idx_ref[...]
    def gath(comps_ref, out_ref):
        rb = out_ref.shape[0]
        comps = comps_ref[...]
        x3 = jnp.broadcast_to(comps[:, None, :], (rb, 8, 64))
        for o in range(16):
            idx3 = jnp.broadcast_to(idx[None, 8 * o:8 * (o + 1), :],
                                    (rb, 8, 128))
            out_ref[:, 8 * o:8 * (o + 1), :] = (
                jnp.take_along_axis(x3, idx3, axis=2))
    gath(k1_ref, o1_ref)
    gath(kv1_ref, o11_ref)
    ct32 = ct32_ref[...]
    ct16 = ct16_ref[...]
    _expand(oh32_ref, ct32[:, :16 * _B], o2_ref)
    _expand(oh16_ref, ct16[:, :32 * _B], o3_ref)
    _expand(oh32_ref, ct32[:, 16 * _B:], o22_ref)
    _expand(oh16_ref, ct16[:, 32 * _B:], o33_ref)


def _full_spec(rows, cols):
    return pl.BlockSpec((rows, cols), lambda i: (0, 0))


def kernel(x, W1, b1, g1, beta1, W2, b2, g2, beta2, W3, b3, g3, beta3):
    f32 = jnp.float32

    def vec_spec(n):
        return pl.BlockSpec((n,), lambda i: (0,))

    kv, kvb = pl.pallas_call(
        _mlp_body,
        grid=(_G_MLP,),
        in_specs=[
            pl.BlockSpec((_B // _G_MLP, 1), lambda i: (i, 0)),
            _full_spec(2 * _F, _H1),
            vec_spec(_H1), vec_spec(_H1), vec_spec(_H1),
            _full_spec(_H1, _D1),
            vec_spec(_D1), vec_spec(_D1), vec_spec(_D1),
            _full_spec(_D1, _D2),
            vec_spec(_D2), vec_spec(_D2), vec_spec(_D2),
        ],
        out_specs=[
            pl.BlockSpec((_B // _G_MLP, _D1), lambda i: (i, 0)),
            pl.BlockSpec((_B // _G_MLP, _D2), lambda i: (i, 0)),
        ],
        out_shape=[
            jax.ShapeDtypeStruct((_B, _D1), f32),
            jax.ShapeDtypeStruct((_B, _D2), f32),
        ],
        compiler_params=pltpu.CompilerParams(
            dimension_semantics=("parallel",),
        ),
        name="distance_mlp",
    )(x, W1, b1, g1, beta1,
      W2, b2, g2, beta2,
      W3, b3, g3, beta3)

    def comps_t(mat, n_ch, length):
        return (mat.reshape(_B, n_ch, length)
                   .transpose(2, 1, 0)
                   .reshape(length, n_ch * _B))

    ct32 = jnp.concatenate(
        [comps_t(kvb[:, 512:1024], 16, 32), comps_t(kv[:, 256:512], 8, 32)],
        axis=1)
    ct16 = jnp.concatenate(
        [comps_t(kvb[:, 1024:], 32, 16), comps_t(kv[:, 512:], 16, 16)],
        axis=1)
    kv1r = kv[:, :256].reshape(_B * 4, 64)
    k1r = kvb[:, :512].reshape(_B * 8, 64)

    g = _G_EXP
    r32, r16 = 4096 // g, 1024 // g
    o1, o2, o3, o11, o22, o33 = pl.pallas_call(
        _expand_all_body,
        grid=(g,),
        in_specs=[
            _full_spec(128, 128),
            pl.BlockSpec((r32, 32), lambda i: (i, 0)),
            pl.BlockSpec((r16, 16), lambda i: (i, 0)),
            pl.BlockSpec((_B * 8 // g, 64), lambda i: (i, 0)),
            _full_spec(32, 24 * _B),
            _full_spec(16, 48 * _B),
            pl.BlockSpec((_B * 4 // g, 64), lambda i: (i, 0)),
        ],
        out_specs=[
            pl.BlockSpec((_B * 8 // g, 128, 128), lambda i: (i, 0, 0)),
            pl.BlockSpec((16, 64 // g, 64, _B), lambda i: (0, i, 0, 0)),
            pl.BlockSpec((32, 32 // g, 32, _B), lambda i: (0, i, 0, 0)),
            pl.BlockSpec((_B * 4 // g, 128, 128), lambda i: (i, 0, 0)),
            pl.BlockSpec((8, 64 // g, 64, _B), lambda i: (0, i, 0, 0)),
            pl.BlockSpec((16, 32 // g, 32, _B), lambda i: (0, i, 0, 0)),
        ],
        out_shape=[
            jax.ShapeDtypeStruct((_B * 8, 128, 128), f32),
            jax.ShapeDtypeStruct((16, 64, 64, _B), f32),
            jax.ShapeDtypeStruct((32, 32, 32, _B), f32),
            jax.ShapeDtypeStruct((_B * 4, 128, 128), f32),
            jax.ShapeDtypeStruct((8, 64, 64, _B), f32),
            jax.ShapeDtypeStruct((16, 32, 32, _B), f32),
        ],
        compiler_params=pltpu.CompilerParams(
            dimension_semantics=("parallel",),
            vmem_limit_bytes=56 * 1024 * 1024,
        ),
        name="distance_expand",
    )(_bin_idx(64), _bin_onehot_t(32), _bin_onehot_t(16),
      k1r, ct32, ct16, kv1r)

    def finalize(o):
        return o.transpose(3, 0, 1, 2)

    c1 = o1.reshape(_B, 8, 128, 128)
    c2 = finalize(o2)
    c3 = finalize(o3)
    c11 = o11.reshape(_B, 4, 128, 128)
    c22 = finalize(o22)
    c33 = finalize(o33)
    return (c1, c2, c3, c11, c22, c33)


# speedup vs baseline: 7.7895x; 1.0282x over previous
import jax
import jax.numpy as jnp
from jax.experimental import pallas as pl
from jax.experimental.pallas import tpu as pltpu

_B = 128
_F = 32
_WL = 0.638
_PITCH = 8e-06
_H1 = 512
_D1 = 768
_D2 = 1536

_G_MLP = 2
_G_EXP = 16


import numpy as np

_WAVELENGTH = _WL * 1e-06
_MIN_FRE = 2 * np.pi / _WAVELENGTH * (1 - 2 * (_WAVELENGTH / _PITCH / 2) ** 2) ** 0.5
_MAX_FRE = 2 * np.pi / _WAVELENGTH
_FB_SCALE = np.float32((_MAX_FRE - _MIN_FRE) / _F)
_FB_MIN = np.float32(_MIN_FRE)


def _bin_idx(length):
    ax = jnp.linspace(-float(length), float(length), 2 * length)
    xg, yg = jnp.meshgrid(ax, ax, indexing='ij')
    dis = jnp.sqrt(xg ** 2 + yg ** 2)
    interval = jnp.max(dis) / length
    return jnp.floor(dis / (interval + 0.0001)).astype(jnp.int32)


def _bin_onehot_t(length):
    idx = _bin_idx(length)
    flat = idx.reshape(4 * length * length, 1)
    lanes = jnp.arange(length, dtype=jnp.int32).reshape(1, length)
    return (flat == lanes).astype(jnp.float32)


def _ln_scale(z, g, b):
    m = jnp.mean(z, axis=-1, keepdims=True)
    v = jnp.mean((z - m) ** 2, axis=-1, keepdims=True)
    return (z - m) * jax.lax.rsqrt(v + 1e-5) * g + b


def _mlp_body(x_ref, w1_ref, b1_ref, g1_ref, be1_ref,
              w2_ref, b2_ref, g2_ref, be2_ref,
              w3_ref, b3_ref, g3_ref, be3_ref, kv_ref, kvb_ref):
    bands = (jax.lax.broadcasted_iota(jnp.int32, (1, _F), 1) + 1).astype(jnp.float32)
    fb = bands * _FB_SCALE + _FB_MIN
    ds = x_ref[...] * fb
    z1 = (jnp.dot(jnp.cos(ds), w1_ref[:_F, :], preferred_element_type=jnp.float32)
          + jnp.dot(jnp.sin(ds), w1_ref[_F:, :], preferred_element_type=jnp.float32)
          + b1_ref[...])
    h = jnp.tanh(_ln_scale(z1, g1_ref[...], be1_ref[...]))
    z2 = jnp.dot(h, w2_ref[...], preferred_element_type=jnp.float32) + b2_ref[...]
    kv = jnp.tanh(_ln_scale(z2, g2_ref[...], be2_ref[...]))
    kv_ref[...] = kv
    z3 = jnp.dot(kv, w3_ref[...], preferred_element_type=jnp.float32) + b3_ref[...]
    kvb_ref[...] = jnp.tanh(_ln_scale(z3, g3_ref[...], be3_ref[...]))


def _expand(oh_ref, ct, out_ref):
    n_ch, di = out_ref.shape[0], out_ref.shape[1]
    t = jnp.dot(oh_ref[...], ct,
                preferred_element_type=jnp.float32)
    for c in range(n_ch):
        out_ref[c] = t[:, c * _B:(c + 1) * _B].reshape(di, -1, _B)


def _expand_all_body(idx_ref, oh32_ref, oh16_ref,
                     k1_ref, ct32_ref, ct16_ref, kv1_ref,
                     o1_ref, o2_ref, o3_ref, o11_ref, o22_ref, o33_ref):
    idx = idx_ref[...]
    def gath(comps_ref, out_ref):
        rb = out_ref.shape[0]
        comps = comps_ref[...]
        x3 = jnp.broadcast_to(comps[:, None, :], (rb, 8, 64))
        for o in range(16):
            idx3 = jnp.broadcast_to(idx[None, 8 * o:8 * (o + 1), :],
                                    (rb, 8, 128))
            out_ref[:, 8 * o:8 * (o + 1), :] = (
                jnp.take_along_axis(x3, idx3, axis=2))
    gath(k1_ref, o1_ref)
    gath(kv1_ref, o11_ref)
    ct32 = ct32_ref[...]
    ct16 = ct16_ref[...]
    _expand(oh32_ref, ct32[:, :16 * _B], o2_ref)
    _expand(oh16_ref, ct16[:, :32 * _B], o3_ref)
    _expand(oh32_ref, ct32[:, 16 * _B:], o22_ref)
    _expand(oh16_ref, ct16[:, 32 * _B:], o33_ref)


def _full_spec(rows, cols):
    return pl.BlockSpec((rows, cols), lambda i: (0, 0))


def kernel(x, W1, b1, g1, beta1, W2, b2, g2, beta2, W3, b3, g3, beta3):
    f32 = jnp.float32

    def vec_spec(n):
        return pl.BlockSpec((n,), lambda i: (0,))

    kv, kvb = pl.pallas_call(
        _mlp_body,
        grid=(_G_MLP,),
        in_specs=[
            pl.BlockSpec((_B // _G_MLP, 1), lambda i: (i, 0)),
            _full_spec(2 * _F, _H1),
            vec_spec(_H1), vec_spec(_H1), vec_spec(_H1),
            _full_spec(_H1, _D1),
            vec_spec(_D1), vec_spec(_D1), vec_spec(_D1),
            _full_spec(_D1, _D2),
            vec_spec(_D2), vec_spec(_D2), vec_spec(_D2),
        ],
        out_specs=[
            pl.BlockSpec((_B // _G_MLP, _D1), lambda i: (i, 0)),
            pl.BlockSpec((_B // _G_MLP, _D2), lambda i: (i, 0)),
        ],
        out_shape=[
            jax.ShapeDtypeStruct((_B, _D1), f32),
            jax.ShapeDtypeStruct((_B, _D2), f32),
        ],
        compiler_params=pltpu.CompilerParams(
            dimension_semantics=("parallel",),
        ),
        name="distance_mlp",
    )(x, W1, b1, g1, beta1,
      W2, b2, g2, beta2,
      W3, b3, g3, beta3)

    def comps_t(mat, n_ch, length):
        return (mat.reshape(_B, n_ch, length)
                   .transpose(2, 1, 0)
                   .reshape(length, n_ch * _B))

    ct32 = jnp.concatenate(
        [comps_t(kvb[:, 512:1024], 16, 32), comps_t(kv[:, 256:512], 8, 32)],
        axis=1)
    ct16 = jnp.concatenate(
        [comps_t(kvb[:, 1024:], 32, 16), comps_t(kv[:, 512:], 16, 16)],
        axis=1)
    kv1r = kv[:, :256].reshape(_B * 4, 64)
    k1r = kvb[:, :512].reshape(_B * 8, 64)

    g = _G_EXP
    r32, r16 = 4096 // g, 1024 // g
    o1, o2, o3, o11, o22, o33 = pl.pallas_call(
        _expand_all_body,
        grid=(g,),
        in_specs=[
            _full_spec(128, 128),
            pl.BlockSpec((r32, 32), lambda i: (i, 0)),
            pl.BlockSpec((r16, 16), lambda i: (i, 0)),
            pl.BlockSpec((_B * 8 // g, 64), lambda i: (i, 0)),
            _full_spec(32, 24 * _B),
            _full_spec(16, 48 * _B),
            pl.BlockSpec((_B * 4 // g, 64), lambda i: (i, 0)),
        ],
        out_specs=[
            pl.BlockSpec((_B * 8 // g, 128, 128), lambda i: (i, 0, 0)),
            pl.BlockSpec((16, 64 // g, 64, _B), lambda i: (0, i, 0, 0)),
            pl.BlockSpec((32, 32 // g, 32, _B), lambda i: (0, i, 0, 0)),
            pl.BlockSpec((_B * 4 // g, 128, 128), lambda i: (i, 0, 0)),
            pl.BlockSpec((8, 64 // g, 64, _B), lambda i: (0, i, 0, 0)),
            pl.BlockSpec((16, 32 // g, 32, _B), lambda i: (0, i, 0, 0)),
        ],
        out_shape=[
            jax.ShapeDtypeStruct((_B * 8, 128, 128), f32),
            jax.ShapeDtypeStruct((16, 64, 64, _B), f32),
            jax.ShapeDtypeStruct((32, 32, 32, _B), f32),
            jax.ShapeDtypeStruct((_B * 4, 128, 128), f32),
            jax.ShapeDtypeStruct((8, 64, 64, _B), f32),
            jax.ShapeDtypeStruct((16, 32, 32, _B), f32),
        ],
        compiler_params=pltpu.CompilerParams(
            dimension_semantics=("parallel",),
            vmem_limit_bytes=56 * 1024 * 1024,
        ),
        name="distance_expand",
    )(_bin_idx(64), _bin_onehot_t(32), _bin_onehot_t(16),
      k1r, ct32, ct16, kv1r)

    def finalize(o):
        return o.transpose(3, 0, 1, 2)

    c1 = o1.reshape(_B, 8, 128, 128)
    c2 = finalize(o2)
    c3 = finalize(o3)
    c11 = o11.reshape(_B, 4, 128, 128)
    c22 = finalize(o22)
    c33 = finalize(o33)
    return (c1, c2, c3, c11, c22, c33)
